# Initial kernel scaffold; baseline (speedup 1.0000x reference)
import functools

import jax
import jax.numpy as jnp
from jax import lax
from jax.experimental import pallas as pl
from jax.experimental.pallas import tpu as pltpu
from jax.experimental.pallas import tpu_sc as plsc

NC, NS = 2, 16
ECHUNK = 128


def _ceil_to(x, m):
    return (x + m - 1) // m * m


def _tca_body(eps, f_ref, c_ref, wpre_ref, g_ref, b_ref, wpos_ref, alpha_ref,
              cat0_ref, cat1_ref, pwc_ref, pws_ref):
    x = jnp.dot(f_ref[...], wpre_ref[...], preferred_element_type=jnp.float32)
    m = jnp.mean(x, axis=-1, keepdims=True)
    v = jnp.mean((x - m) ** 2, axis=-1, keepdims=True)
    fi = (x - m) / jnp.sqrt(v + eps) * g_ref[...] + b_ref[...]
    pw = jnp.dot(c_ref[...], wpos_ref[...],
                 preferred_element_type=jnp.float32) * alpha_ref[...]
    s = jnp.sin(pw)
    co = jnp.cos(pw)
    fc = fi * co
    fs = fi * s
    fl = fi * pw
    cat0_ref[...] = jnp.concatenate([fc, fs[:, :32]], axis=1)
    cat1_ref[...] = jnp.concatenate([fs[:, 32:], fl], axis=1)
    pwc_ref[...] = co
    pws_ref[...] = s


def _run_tca(Fp, Cp, W_pre, ln_g, ln_b, W_posp, alpha):
    npt, inc = Fp.shape
    blk = 1024
    grid = (npt // blk,)
    row_spec = lambda w: pl.BlockSpec((blk, w), lambda i: (i, 0))
    full2 = lambda a: pl.BlockSpec(a.shape, lambda i: (0,) * a.ndim)
    return pl.pallas_call(
        functools.partial(_tca_body, 1e-6),
        grid=grid,
        in_specs=[row_spec(inc), row_spec(Cp.shape[1]), full2(W_pre),
                  full2(ln_g), full2(ln_b), full2(W_posp), full2(alpha)],
        out_specs=[row_spec(96), row_spec(96), row_spec(inc), row_spec(inc)],
        out_shape=[jax.ShapeDtypeStruct((npt, 96), jnp.float32),
                   jax.ShapeDtypeStruct((npt, 96), jnp.float32),
                   jax.ShapeDtypeStruct((npt, inc), jnp.float32),
                   jax.ShapeDtypeStruct((npt, inc), jnp.float32)],
    )(Fp, Cp, W_pre, ln_g, ln_b, W_posp, alpha)


def _scb_body(nch, npad,
              f0_hbm, f1_hbm, src_hbm, dst_hbm, z32_hbm, z1_hbm, ones_hbm,
              agg0_hbm, agg1_hbm, deg_hbm,
              src_v, dst_v, rows_a, rows_b, ones_v, agg_sh, deg_sh,
              sem_a, sem_b):
    core = lax.axis_index("c")
    sid = lax.axis_index("s")
    rpt = npad // NS

    def run_core(f_hbm, agg_out, do_deg):
        zslc = pl.ds(sid * rpt, rpt)
        pltpu.sync_copy(z32_hbm.at[zslc], agg_sh.at[zslc])
        if do_deg:
            pltpu.sync_copy(z1_hbm.at[zslc], deg_sh.at[zslc])
            pltpu.sync_copy(ones_hbm, ones_v)
        pltpu.sync_copy(src_hbm.at[sid], src_v)
        pltpu.sync_copy(dst_hbm.at[sid], dst_v)
        plsc.subcore_barrier()

        def issue(j, buf, sem):
            pltpu.async_copy(f_hbm.at[src_v.at[j]], buf, sem)

        def wait(j, buf, sem):
            pltpu.make_async_copy(f_hbm.at[src_v.at[j]], buf, sem).wait()

        def scat(j, buf):
            pltpu.sync_copy(buf, agg_sh.at[dst_v.at[j]], add=True)
            if do_deg:
                pltpu.sync_copy(ones_v, deg_sh.at[dst_v.at[j]], add=True)

        issue(0, rows_a, sem_a)

        @pl.loop(0, nch, step=2)
        def _(j):
            issue(j + 1, rows_b, sem_b)
            wait(j, rows_a, sem_a)
            scat(j, rows_a)

            @pl.when(j + 2 < nch)
            def _():
                issue(j + 2, rows_a, sem_a)

            wait(j + 1, rows_b, sem_b)
            scat(j + 1, rows_b)

        plsc.subcore_barrier()
        pltpu.sync_copy(agg_sh.at[zslc], agg_out.at[zslc])
        if do_deg:
            pltpu.sync_copy(deg_sh.at[zslc], deg_hbm.at[zslc])

    @pl.when(core == 0)
    def _():
        run_core(f0_hbm, agg0_hbm, True)

    @pl.when(core == 1)
    def _():
        run_core(f1_hbm, agg1_hbm, False)


def _run_scb(f0, f1, srcp, dstp, npad):
    nch = srcp.shape[1]
    mesh = plsc.VectorSubcoreMesh(core_axis_name="c", subcore_axis_name="s")
    z32 = jnp.zeros((npad, 32), jnp.float32)
    z1 = jnp.zeros((npad, 1), jnp.float32)
    ones = jnp.ones((ECHUNK, 1), jnp.float32)
    kern = pl.kernel(
        functools.partial(_scb_body, nch, npad),
        out_type=[jax.ShapeDtypeStruct((npad, 32), jnp.float32),
                  jax.ShapeDtypeStruct((npad, 32), jnp.float32),
                  jax.ShapeDtypeStruct((npad, 1), jnp.float32)],
        mesh=mesh,
        scratch_types=[
            pltpu.VMEM((nch, ECHUNK), jnp.int32),
            pltpu.VMEM((nch, ECHUNK), jnp.int32),
            pltpu.VMEM((ECHUNK, 32), jnp.float32),
            pltpu.VMEM((ECHUNK, 32), jnp.float32),
            pltpu.VMEM((ECHUNK, 1), jnp.float32),
            pltpu.VMEM_SHARED((npad, 32), jnp.float32),
            pltpu.VMEM_SHARED((npad, 1), jnp.float32),
            pltpu.SemaphoreType.DMA,
            pltpu.SemaphoreType.DMA,
        ],
    )
    return kern(f0, f1, srcp, dstp, z32, z1, ones)


def _scc_body(pch, vpad,
              cat0_hbm, cat1_hbm, vox_hbm, zv96_hbm, zv1_hbm, ones_hbm,
              g0_hbm, g1_hbm, gcnt_hbm,
              vox_v, buf_a, buf_b, cnt_buf, ones_v, sums_sh, cnt_sh,
              sem_a, sem_b):
    core = lax.axis_index("c")
    sid = lax.axis_index("s")
    vrpt = vpad // NS

    def run_core(cat_hbm, g_out, do_cnt):
        zslc = pl.ds(sid * vrpt, vrpt)
        pltpu.sync_copy(zv96_hbm.at[zslc], sums_sh.at[zslc])
        if do_cnt:
            pltpu.sync_copy(zv1_hbm.at[zslc], cnt_sh.at[zslc])
            pltpu.sync_copy(ones_hbm, ones_v)
        pltpu.sync_copy(vox_hbm.at[sid], vox_v)
        plsc.subcore_barrier()

        def rows(j):
            return pl.ds((sid * pch + j) * ECHUNK, ECHUNK)

        def issue(j, buf, sem):
            pltpu.async_copy(cat_hbm.at[rows(j)], buf, sem)

        def wait(j, buf, sem):
            pltpu.make_async_copy(cat_hbm.at[rows(j)], buf, sem).wait()

        def scat(j, buf):
            pltpu.sync_copy(buf, sums_sh.at[vox_v.at[j]], add=True)
            if do_cnt:
                pltpu.sync_copy(ones_v, cnt_sh.at[vox_v.at[j]], add=True)

        issue(0, buf_a, sem_a)

        @pl.loop(0, pch, step=2)
        def _(j):
            issue(j + 1, buf_b, sem_b)
            wait(j, buf_a, sem_a)
            scat(j, buf_a)

            @pl.when(j + 2 < pch)
            def _():
                issue(j + 2, buf_a, sem_a)

            wait(j + 1, buf_b, sem_b)
            scat(j + 1, buf_b)

        plsc.subcore_barrier()

        @pl.loop(0, pch)
        def _(j):
            pltpu.sync_copy(sums_sh.at[vox_v.at[j]], buf_a)
            pltpu.sync_copy(buf_a, g_out.at[rows(j)])
            if do_cnt:
                pltpu.sync_copy(cnt_sh.at[vox_v.at[j]], cnt_buf)
                pltpu.sync_copy(cnt_buf, gcnt_hbm.at[rows(j)])

    @pl.when(core == 0)
    def _():
        run_core(cat0_hbm, g0_hbm, True)

    @pl.when(core == 1)
    def _():
        run_core(cat1_hbm, g1_hbm, False)


def _run_scc(cat0, cat1, voxp, vpad):
    npt = cat0.shape[0]
    pch = voxp.shape[1]
    mesh = plsc.VectorSubcoreMesh(core_axis_name="c", subcore_axis_name="s")
    zv96 = jnp.zeros((vpad, 96), jnp.float32)
    zv1 = jnp.zeros((vpad, 1), jnp.float32)
    ones = jnp.ones((ECHUNK, 1), jnp.float32)
    kern = pl.kernel(
        functools.partial(_scc_body, pch, vpad),
        out_type=[jax.ShapeDtypeStruct((npt, 96), jnp.float32),
                  jax.ShapeDtypeStruct((npt, 96), jnp.float32),
                  jax.ShapeDtypeStruct((npt, 1), jnp.float32)],
        mesh=mesh,
        scratch_types=[
            pltpu.VMEM((pch, ECHUNK), jnp.int32),
            pltpu.VMEM((ECHUNK, 96), jnp.float32),
            pltpu.VMEM((ECHUNK, 96), jnp.float32),
            pltpu.VMEM((ECHUNK, 1), jnp.float32),
            pltpu.VMEM((ECHUNK, 1), jnp.float32),
            pltpu.VMEM_SHARED((vpad, 96), jnp.float32),
            pltpu.VMEM_SHARED((vpad, 1), jnp.float32),
            pltpu.SemaphoreType.DMA,
            pltpu.SemaphoreType.DMA,
        ],
    )
    return kern(cat0, cat1, voxp, zv96, zv1, ones)


def _tcd_body(eps, g0_ref, g1_ref, gcnt_ref, pwc_ref, pws_ref, cat1_ref,
              agg0_ref, agg1_ref, deg_ref, wl_ref, ng_ref, nb_ref, lg_ref,
              lb_ref, out_ref):
    inv = 1.0 / jnp.maximum(gcnt_ref[...], 1.0)
    g0 = g0_ref[...]
    g1 = g1_ref[...]
    gcos = g0[:, :64] * inv
    gsin = jnp.concatenate([g0[:, 64:], g1[:, :32]], axis=1) * inv
    glin = g1[:, 32:] * inv
    fwl = cat1_ref[...][:, 32:]
    new_f = gcos * pwc_ref[...] + gsin * pws_ref[...] + (glin - fwl)

    agg = jnp.concatenate([agg0_ref[...], agg1_ref[...]], axis=1)
    loc = jnp.dot(agg / jnp.maximum(deg_ref[...], 1.0), wl_ref[...],
                  preferred_element_type=jnp.float32)

    def ln(x, g, b):
        m = jnp.mean(x, axis=-1, keepdims=True)
        v = jnp.mean((x - m) ** 2, axis=-1, keepdims=True)
        return (x - m) / jnp.sqrt(v + eps) * g + b

    out_ref[...] = jax.nn.relu(ln(new_f, ng_ref[...], nb_ref[...])
                               + ln(loc, lg_ref[...], lb_ref[...]))


def _run_tcd(n, g0, g1, gcnt, pwc, pws, cat1, agg0, agg1, deg,
             W_local, norm_g, norm_b, nl_g, nl_b):
    blk = 2000
    grid = (n // blk,)
    row_spec = lambda w: pl.BlockSpec((blk, w), lambda i: (i, 0))
    full2 = lambda a: pl.BlockSpec(a.shape, lambda i: (0,) * a.ndim)
    return pl.pallas_call(
        functools.partial(_tcd_body, 1e-6),
        grid=grid,
        in_specs=[row_spec(96), row_spec(96), row_spec(1), row_spec(64),
                  row_spec(64), row_spec(96), row_spec(32), row_spec(32),
                  row_spec(1), full2(W_local), full2(norm_g), full2(norm_b),
                  full2(nl_g), full2(nl_b)],
        out_specs=row_spec(64),
        out_shape=jax.ShapeDtypeStruct((n, 64), jnp.float32),
    )(g0, g1, gcnt, pwc, pws, cat1, agg0, agg1, deg,
      W_local, norm_g, norm_b, nl_g, nl_b)


def kernel(F, C, edge_index, voxel_idx, W_pre, ln_pre_g, ln_pre_b, W_pos,
           alpha, W_local, norm_g, norm_b, nl_g, nl_b):
    n, inc = F.shape
    e = edge_index.shape[1]
    nvox = 6250

    e_pad = _ceil_to(e, NS * 2 * ECHUNK)
    ept = e_pad // NS
    nch = ept // ECHUNK
    npad = _ceil_to(n + 1, NS * 8)
    pch = _ceil_to(_ceil_to(n, NS * ECHUNK) // (NS * ECHUNK), 2)
    npt = NS * pch * ECHUNK
    vpad = _ceil_to(nvox + 1, NS * 8)

    Fp = jnp.pad(F, ((0, npt - n), (0, 0)))
    Cp = jnp.pad(C, ((0, npt - n), (0, 5)))
    W_posp = jnp.pad(W_pos, ((0, 5), (0, 0)))
    f0 = F[:, :32]
    f1 = F[:, 32:]
    srcp = jnp.concatenate(
        [edge_index[0], jnp.zeros((e_pad - e,), jnp.int32)]).reshape(
            NS, nch, ECHUNK)
    dstp = jnp.concatenate(
        [edge_index[1], jnp.full((e_pad - e,), n, jnp.int32)]).reshape(
            NS, nch, ECHUNK)
    voxp = jnp.concatenate(
        [voxel_idx, jnp.full((npt - n,), nvox, jnp.int32)]).reshape(
            NS, pch, ECHUNK)

    cat0, cat1, pwc, pws = _run_tca(Fp, Cp, W_pre, ln_pre_g, ln_pre_b,
                                    W_posp, alpha)
    agg0, agg1, deg = _run_scb(f0, f1, srcp, dstp, npad)
    g0, g1, gcnt = _run_scc(cat0, cat1, voxp, vpad)
    out = _run_tcd(n, g0, g1, gcnt, pwc, pws, cat1, agg0, agg1, deg,
                   W_local, norm_g, norm_b, nl_g, nl_b)
    return out


# speedup vs baseline: 4.9064x; 4.9064x over previous
import functools

import jax
import jax.numpy as jnp
from jax import lax
from jax.experimental import pallas as pl
from jax.experimental.pallas import tpu as pltpu
from jax.experimental.pallas import tpu_sc as plsc

NC, NS = 2, 16
ECHUNK = 128

_SC_PARAMS = pltpu.CompilerParams(use_tc_tiling_on_sc=False)


def _ceil_to(x, m):
    return (x + m - 1) // m * m


def _tca_body(eps, f_ref, c_ref, wpre_ref, g_ref, b_ref, wpos_ref, alpha_ref,
              cat_ref, pwc_ref, pws_ref):
    x = jnp.dot(f_ref[...], wpre_ref[...], preferred_element_type=jnp.float32)
    m = jnp.mean(x, axis=-1, keepdims=True)
    v = jnp.mean((x - m) ** 2, axis=-1, keepdims=True)
    fi = (x - m) / jnp.sqrt(v + eps) * g_ref[...] + b_ref[...]
    pw = jnp.dot(c_ref[...], wpos_ref[...],
                 preferred_element_type=jnp.float32) * alpha_ref[...]
    s = jnp.sin(pw)
    co = jnp.cos(pw)
    fc = fi * co
    fs = fi * s
    fl = fi * pw
    cat_ref[0] = jnp.concatenate([fc, fs[:, :32]], axis=1)
    cat_ref[1] = jnp.concatenate([fs[:, 32:], fl], axis=1)
    pwc_ref[...] = co
    pws_ref[...] = s


def _run_tca(Fp, Cp, W_pre, ln_g, ln_b, W_posp, alpha):
    npt, inc = Fp.shape
    blk = 1024
    grid = (npt // blk,)
    row_spec = lambda w: pl.BlockSpec((blk, w), lambda i: (i, 0))
    full2 = lambda a: pl.BlockSpec(a.shape, lambda i: (0,) * a.ndim)
    return pl.pallas_call(
        functools.partial(_tca_body, 1e-6),
        grid=grid,
        in_specs=[row_spec(inc), row_spec(Cp.shape[1]), full2(W_pre),
                  full2(ln_g), full2(ln_b), full2(W_posp), full2(alpha)],
        out_specs=[pl.BlockSpec((2, blk, 96), lambda i: (0, i, 0)),
                   row_spec(inc), row_spec(inc)],
        out_shape=[jax.ShapeDtypeStruct((2, npt, 96), jnp.float32),
                   jax.ShapeDtypeStruct((npt, inc), jnp.float32),
                   jax.ShapeDtypeStruct((npt, inc), jnp.float32)],
    )(Fp, Cp, W_pre, ln_g, ln_b, W_posp, alpha)


def _fill(buf, value):
    w = buf.shape[1]

    @pl.loop(0, ECHUNK)
    def _(i):
        row = buf.at[i]
        for k in range(w // 16):
            row[pl.ds(k * 16, 16)] = jnp.full((16,), value, jnp.float32)


def _zero_shared(sh, zbuf, base, nchunks):
    @pl.loop(0, nchunks)
    def _(i):
        pltpu.sync_copy(zbuf, sh.at[pl.ds(base + i * ECHUNK, ECHUNK)])


IBLK = 56


def _scb_body(nch, npad,
              fq_hbm, src_hbm, dst_hbm,
              agg_hbm, deg_hbm,
              src_v, dst_v, rows_a, rows_b, zbuf, ones_v, agg_sh,
              sem_a, sem_b):
    core = lax.axis_index("c")
    sid = lax.axis_index("s")
    rpt = npad // NS
    zslc = pl.ds(sid * rpt, rpt)
    nblk = nch // IBLK
    src_t = src_hbm.at[sid]
    dst_t = dst_hbm.at[sid]

    _fill(zbuf, 0.0)
    _fill(ones_v, 1.0)

    for p in range(2):
        q = 2 * p + core
        f_hbm = fq_hbm.at[q]
        _zero_shared(agg_sh, zbuf, sid * rpt, rpt // ECHUNK)
        plsc.subcore_barrier()

        def issue(j, buf, sem):
            pltpu.async_copy(f_hbm.at[src_v.at[j]], buf, sem)

        def wait(j, buf, sem):
            pltpu.make_async_copy(f_hbm.at[src_v.at[j]], buf, sem).wait()

        def scat(j, buf):
            pltpu.sync_copy(buf, agg_sh.at[dst_v.at[j]], add=True)

        @pl.loop(0, nblk)
        def _(b):
            pltpu.sync_copy(src_t.at[pl.ds(b * IBLK, IBLK)], src_v)
            pltpu.sync_copy(dst_t.at[pl.ds(b * IBLK, IBLK)], dst_v)
            issue(0, rows_a, sem_a)

            @pl.loop(0, IBLK, step=2)
            def _(j):
                issue(j + 1, rows_b, sem_b)
                wait(j, rows_a, sem_a)
                scat(j, rows_a)

                @pl.when(j + 2 < IBLK)
                def _():
                    issue(j + 2, rows_a, sem_a)

                wait(j + 1, rows_b, sem_b)
                scat(j + 1, rows_b)

        plsc.subcore_barrier()
        pltpu.sync_copy(agg_sh.at[zslc], agg_hbm.at[q].at[zslc])

    _zero_shared(agg_sh, zbuf, sid * rpt, rpt // ECHUNK)
    plsc.subcore_barrier()

    @pl.loop(0, nblk)
    def _(b):
        pltpu.sync_copy(dst_t.at[pl.ds(b * IBLK, IBLK)], dst_v)

        @pl.loop(0, IBLK)
        def _(j):
            pltpu.sync_copy(ones_v, agg_sh.at[dst_v.at[j]], add=True)

    plsc.subcore_barrier()
    pltpu.sync_copy(agg_sh.at[zslc], deg_hbm.at[zslc])


def _run_scb(fq, srcp, dstp, npad):
    nch = srcp.shape[1]
    mesh = plsc.VectorSubcoreMesh(core_axis_name="c", subcore_axis_name="s")
    kern = pl.kernel(
        functools.partial(_scb_body, nch, npad),
        out_type=[jax.ShapeDtypeStruct((4, npad, 16), jnp.float32),
                  jax.ShapeDtypeStruct((npad, 16), jnp.float32)],
        mesh=mesh,
        scratch_types=[
            pltpu.VMEM((IBLK, ECHUNK), jnp.int32),
            pltpu.VMEM((IBLK, ECHUNK), jnp.int32),
            pltpu.VMEM((ECHUNK, 16), jnp.float32),
            pltpu.VMEM((ECHUNK, 16), jnp.float32),
            pltpu.VMEM((ECHUNK, 16), jnp.float32),
            pltpu.VMEM((ECHUNK, 16), jnp.float32),
            pltpu.VMEM_SHARED((npad, 16), jnp.float32),
            pltpu.SemaphoreType.DMA,
            pltpu.SemaphoreType.DMA,
        ],
        compiler_params=_SC_PARAMS,
    )
    return kern(fq, srcp, dstp)


def _scc_body(pch, vpad,
              cat_hbm, vox_hbm,
              g_hbm, gcnt_hbm,
              vox_v, buf_a, buf_b, cnt_buf, zbuf96, ones_v, sums_sh, cnt_sh,
              sem_a, sem_b):
    core = lax.axis_index("c")
    sid = lax.axis_index("s")
    vrpt = vpad // NS
    cat_c = cat_hbm.at[core]
    g_c = g_hbm.at[core]

    _fill(zbuf96, 0.0)
    _fill(cnt_buf, 0.0)
    _fill(ones_v, 1.0)
    _zero_shared(sums_sh, zbuf96, sid * vrpt, vrpt // ECHUNK)
    _zero_shared(cnt_sh, cnt_buf, sid * vrpt, vrpt // ECHUNK)
    pltpu.sync_copy(vox_hbm.at[sid], vox_v)
    plsc.subcore_barrier()

    def rows(j):
        return pl.ds((sid * pch + j) * ECHUNK, ECHUNK)

    def issue(j, buf, sem):
        pltpu.async_copy(cat_c.at[rows(j)], buf, sem)

    def wait(j, buf, sem):
        pltpu.make_async_copy(cat_c.at[rows(j)], buf, sem).wait()

    def scat(j, buf):
        pltpu.sync_copy(buf, sums_sh.at[vox_v.at[j]], add=True)
        pltpu.sync_copy(ones_v, cnt_sh.at[vox_v.at[j]], add=True)

    issue(0, buf_a, sem_a)

    @pl.loop(0, pch, step=2)
    def _(j):
        issue(j + 1, buf_b, sem_b)
        wait(j, buf_a, sem_a)
        scat(j, buf_a)

        @pl.when(j + 2 < pch)
        def _():
            issue(j + 2, buf_a, sem_a)

        wait(j + 1, buf_b, sem_b)
        scat(j + 1, buf_b)

    plsc.subcore_barrier()

    @pl.loop(0, pch)
    def _(j):
        pltpu.sync_copy(sums_sh.at[vox_v.at[j]], buf_a)
        pltpu.sync_copy(buf_a, g_c.at[rows(j)])
        pltpu.sync_copy(cnt_sh.at[vox_v.at[j]], cnt_buf)
        pltpu.sync_copy(cnt_buf, gcnt_hbm.at[rows(j)])


def _run_scc(cat_all, voxp, vpad):
    npt = cat_all.shape[1]
    pch = voxp.shape[1]
    mesh = plsc.VectorSubcoreMesh(core_axis_name="c", subcore_axis_name="s")
    kern = pl.kernel(
        functools.partial(_scc_body, pch, vpad),
        out_type=[jax.ShapeDtypeStruct((2, npt, 96), jnp.float32),
                  jax.ShapeDtypeStruct((npt, 16), jnp.float32)],
        mesh=mesh,
        scratch_types=[
            pltpu.VMEM((pch, ECHUNK), jnp.int32),
            pltpu.VMEM((ECHUNK, 96), jnp.float32),
            pltpu.VMEM((ECHUNK, 96), jnp.float32),
            pltpu.VMEM((ECHUNK, 16), jnp.float32),
            pltpu.VMEM((ECHUNK, 96), jnp.float32),
            pltpu.VMEM((ECHUNK, 16), jnp.float32),
            pltpu.VMEM_SHARED((vpad, 96), jnp.float32),
            pltpu.VMEM_SHARED((vpad, 16), jnp.float32),
            pltpu.SemaphoreType.DMA,
            pltpu.SemaphoreType.DMA,
        ],
        compiler_params=_SC_PARAMS,
    )
    return kern(cat_all, voxp)


def _tcd_body(eps, gall_ref, gcnt_ref, pwc_ref, pws_ref, cat1_ref,
              agg_ref, deg_ref, wl_ref, ng_ref, nb_ref, lg_ref,
              lb_ref, out_ref):
    inv = 1.0 / jnp.maximum(gcnt_ref[...][:, :1], 1.0)
    g0 = gall_ref[0]
    g1 = gall_ref[1]
    gcos = g0[:, :64] * inv
    gsin = jnp.concatenate([g0[:, 64:], g1[:, :32]], axis=1) * inv
    glin = g1[:, 32:] * inv
    fwl = cat1_ref[0][:, 32:]
    new_f = gcos * pwc_ref[...] + gsin * pws_ref[...] + (glin - fwl)

    a = agg_ref[...]
    agg = jnp.concatenate([a[0], a[1], a[2], a[3]], axis=1)
    loc = jnp.dot(agg / jnp.maximum(deg_ref[...][:, :1], 1.0), wl_ref[...],
                  preferred_element_type=jnp.float32)

    def ln(x, g, b):
        m = jnp.mean(x, axis=-1, keepdims=True)
        v = jnp.mean((x - m) ** 2, axis=-1, keepdims=True)
        return (x - m) / jnp.sqrt(v + eps) * g + b

    out_ref[...] = jax.nn.relu(ln(new_f, ng_ref[...], nb_ref[...])
                               + ln(loc, lg_ref[...], lb_ref[...]))


def _run_tcd(n, g_all, gcnt, pwc, pws, cat_all, agg_all, deg,
             W_local, norm_g, norm_b, nl_g, nl_b):
    blk = 2000
    grid = (n // blk,)
    row_spec = lambda w: pl.BlockSpec((blk, w), lambda i: (i, 0))
    full2 = lambda a: pl.BlockSpec(a.shape, lambda i: (0,) * a.ndim)
    return pl.pallas_call(
        functools.partial(_tcd_body, 1e-6),
        grid=grid,
        in_specs=[pl.BlockSpec((2, blk, 96), lambda i: (0, i, 0)),
                  row_spec(16), row_spec(64), row_spec(64),
                  pl.BlockSpec((1, blk, 96), lambda i: (1, i, 0)),
                  pl.BlockSpec((4, blk, 16), lambda i: (0, i, 0)),
                  row_spec(16), full2(W_local), full2(norm_g), full2(norm_b),
                  full2(nl_g), full2(nl_b)],
        out_specs=row_spec(64),
        out_shape=jax.ShapeDtypeStruct((n, 64), jnp.float32),
    )(g_all, gcnt, pwc, pws, cat_all, agg_all, deg,
      W_local, norm_g, norm_b, nl_g, nl_b)


def kernel(F, C, edge_index, voxel_idx, W_pre, ln_pre_g, ln_pre_b, W_pos,
           alpha, W_local, norm_g, norm_b, nl_g, nl_b):
    n, inc = F.shape
    e = edge_index.shape[1]
    nvox = 6250

    e_pad = _ceil_to(e, NS * IBLK * ECHUNK)
    ept = e_pad // NS
    nch = ept // ECHUNK
    npad = _ceil_to(n + 1, NS * ECHUNK)
    pch = _ceil_to(_ceil_to(n, NS * ECHUNK) // (NS * ECHUNK), 2)
    npt = NS * pch * ECHUNK
    vpad = _ceil_to(nvox + 1, NS * ECHUNK)

    Fp = jnp.pad(F, ((0, npt - n), (0, 0)))
    Cp = jnp.pad(C, ((0, npt - n), (0, 5)))
    W_posp = jnp.pad(W_pos, ((0, 5), (0, 0)))
    fq = jnp.transpose(F.reshape(n, 4, 16), (1, 0, 2))
    srcp = jnp.concatenate(
        [edge_index[0], jnp.zeros((e_pad - e,), jnp.int32)]).reshape(
            NS, nch, ECHUNK)
    dstp = jnp.concatenate(
        [edge_index[1], jnp.full((e_pad - e,), n, jnp.int32)]).reshape(
            NS, nch, ECHUNK)
    voxp = jnp.concatenate(
        [voxel_idx, jnp.full((npt - n,), nvox, jnp.int32)]).reshape(
            NS, pch, ECHUNK)

    cat_all, pwc, pws = _run_tca(Fp, Cp, W_pre, ln_pre_g, ln_pre_b,
                                 W_posp, alpha)
    agg_all, deg = _run_scb(fq, srcp, dstp, npad)
    g_all, gcnt = _run_scc(cat_all, voxp, vpad)
    out = _run_tcd(n, g_all, gcnt, pwc, pws, cat_all, agg_all, deg,
                   W_local, norm_g, norm_b, nl_g, nl_b)
    return out


# speedup vs baseline: 4.9133x; 1.0014x over previous
import functools

import jax
import jax.numpy as jnp
from jax import lax
from jax.experimental import pallas as pl
from jax.experimental.pallas import tpu as pltpu
from jax.experimental.pallas import tpu_sc as plsc

NC, NS = 2, 16
ECHUNK = 128

_SC_PARAMS = pltpu.CompilerParams(use_tc_tiling_on_sc=False)


def _ceil_to(x, m):
    return (x + m - 1) // m * m


def _tca_body(eps, f_ref, c_ref, wpre_ref, g_ref, b_ref, wpos_ref, alpha_ref,
              cat_ref, pwc_ref, pws_ref):
    x = jnp.dot(f_ref[...], wpre_ref[...], preferred_element_type=jnp.float32)
    m = jnp.mean(x, axis=-1, keepdims=True)
    v = jnp.mean((x - m) ** 2, axis=-1, keepdims=True)
    fi = (x - m) / jnp.sqrt(v + eps) * g_ref[...] + b_ref[...]
    pw = jnp.dot(c_ref[...], wpos_ref[...],
                 preferred_element_type=jnp.float32) * alpha_ref[...]
    s = jnp.sin(pw)
    co = jnp.cos(pw)
    fc = fi * co
    fs = fi * s
    fl = fi * pw
    cat_ref[0] = jnp.concatenate([fc, fs[:, :32]], axis=1)
    cat_ref[1] = jnp.concatenate([fs[:, 32:], fl], axis=1)
    pwc_ref[...] = co
    pws_ref[...] = s


def _run_tca(Fp, Cp, W_pre, ln_g, ln_b, W_posp, alpha):
    npt, inc = Fp.shape
    blk = 1024
    grid = (npt // blk,)
    row_spec = lambda w: pl.BlockSpec((blk, w), lambda i: (i, 0))
    full2 = lambda a: pl.BlockSpec(a.shape, lambda i: (0,) * a.ndim)
    return pl.pallas_call(
        functools.partial(_tca_body, 1e-6),
        grid=grid,
        in_specs=[row_spec(inc), row_spec(Cp.shape[1]), full2(W_pre),
                  full2(ln_g), full2(ln_b), full2(W_posp), full2(alpha)],
        out_specs=[pl.BlockSpec((2, blk, 96), lambda i: (0, i, 0)),
                   row_spec(inc), row_spec(inc)],
        out_shape=[jax.ShapeDtypeStruct((2, npt, 96), jnp.float32),
                   jax.ShapeDtypeStruct((npt, inc), jnp.float32),
                   jax.ShapeDtypeStruct((npt, inc), jnp.float32)],
    )(Fp, Cp, W_pre, ln_g, ln_b, W_posp, alpha)


def _fill(buf, value):
    w = buf.shape[1]

    @pl.loop(0, ECHUNK)
    def _(i):
        row = buf.at[i]
        for k in range(w // 16):
            row[pl.ds(k * 16, 16)] = jnp.full((16,), value, jnp.float32)


def _zero_shared(sh, zbuf, base, nchunks):
    @pl.loop(0, nchunks)
    def _(i):
        pltpu.sync_copy(zbuf, sh.at[pl.ds(base + i * ECHUNK, ECHUNK)])


IBLK = 56


def _scb_body(nch, npad,
              fq_hbm, src_hbm, dst_hbm,
              agg_hbm, deg_hbm,
              src_v, dst_v, rows_a, rows_b, zbuf, ones_v, agg_sh,
              sem_ga, sem_gb, sem_sa, sem_sb):
    core = lax.axis_index("c")
    sid = lax.axis_index("s")
    rpt = npad // NS
    zslc = pl.ds(sid * rpt, rpt)
    nblk = nch // IBLK
    src_t = src_hbm.at[sid]
    dst_t = dst_hbm.at[sid]

    _fill(zbuf, 0.0)
    _fill(ones_v, 1.0)

    for p in range(2):
        q = 2 * p + core
        f_hbm = fq_hbm.at[q]
        _zero_shared(agg_sh, zbuf, sid * rpt, rpt // ECHUNK)
        plsc.subcore_barrier()

        def issue_g(j, buf, sem):
            pltpu.async_copy(f_hbm.at[src_v.at[j]], buf, sem)

        def wait_g(j, buf, sem):
            pltpu.make_async_copy(f_hbm.at[src_v.at[j]], buf, sem).wait()

        def issue_s(j, buf, sem):
            pltpu.async_copy(buf, agg_sh.at[dst_v.at[j]], sem, add=True)

        def wait_s(j, buf, sem):
            pltpu.make_async_copy(buf, agg_sh.at[dst_v.at[j]], sem).wait()

        @pl.loop(0, nblk)
        def _(b):
            pltpu.sync_copy(src_t.at[pl.ds(b * IBLK, IBLK)], src_v)
            pltpu.sync_copy(dst_t.at[pl.ds(b * IBLK, IBLK)], dst_v)
            issue_g(0, rows_a, sem_ga)
            issue_g(1, rows_b, sem_gb)

            @pl.loop(0, IBLK, step=2)
            def _(j):
                wait_g(j, rows_a, sem_ga)
                issue_s(j, rows_a, sem_sa)
                wait_g(j + 1, rows_b, sem_gb)
                issue_s(j + 1, rows_b, sem_sb)

                @pl.when(j + 2 < IBLK)
                def _():
                    wait_s(j, rows_a, sem_sa)
                    issue_g(j + 2, rows_a, sem_ga)
                    wait_s(j + 1, rows_b, sem_sb)
                    issue_g(j + 3, rows_b, sem_gb)

            wait_s(IBLK - 2, rows_a, sem_sa)
            wait_s(IBLK - 1, rows_b, sem_sb)

        plsc.subcore_barrier()
        pltpu.sync_copy(agg_sh.at[zslc], agg_hbm.at[q].at[zslc])

    _zero_shared(agg_sh, zbuf, sid * rpt, rpt // ECHUNK)
    plsc.subcore_barrier()

    @pl.loop(0, nblk)
    def _(b):
        pltpu.sync_copy(dst_t.at[pl.ds(b * IBLK, IBLK)], dst_v)

        @pl.loop(0, IBLK, step=8)
        def _(j):
            for k in range(8):
                pltpu.async_copy(ones_v, agg_sh.at[dst_v.at[j + k]], sem_sa,
                                 add=True)
            for k in range(8):
                pltpu.make_async_copy(ones_v, agg_sh.at[dst_v.at[j + k]],
                                      sem_sa).wait()

    plsc.subcore_barrier()
    pltpu.sync_copy(agg_sh.at[zslc], deg_hbm.at[zslc])


def _run_scb(fq, srcp, dstp, npad):
    nch = srcp.shape[1]
    mesh = plsc.VectorSubcoreMesh(core_axis_name="c", subcore_axis_name="s")
    kern = pl.kernel(
        functools.partial(_scb_body, nch, npad),
        out_type=[jax.ShapeDtypeStruct((4, npad, 16), jnp.float32),
                  jax.ShapeDtypeStruct((npad, 16), jnp.float32)],
        mesh=mesh,
        scratch_types=[
            pltpu.VMEM((IBLK, ECHUNK), jnp.int32),
            pltpu.VMEM((IBLK, ECHUNK), jnp.int32),
            pltpu.VMEM((ECHUNK, 16), jnp.float32),
            pltpu.VMEM((ECHUNK, 16), jnp.float32),
            pltpu.VMEM((ECHUNK, 16), jnp.float32),
            pltpu.VMEM((ECHUNK, 16), jnp.float32),
            pltpu.VMEM_SHARED((npad, 16), jnp.float32),
            pltpu.SemaphoreType.DMA,
            pltpu.SemaphoreType.DMA,
            pltpu.SemaphoreType.DMA,
            pltpu.SemaphoreType.DMA,
        ],
        compiler_params=_SC_PARAMS,
    )
    return kern(fq, srcp, dstp)


def _scc_body(pch, vpad,
              cat_hbm, vox_hbm,
              g_hbm, gcnt_hbm,
              vox_v, buf_a, buf_b, cnt_buf, zbuf96, ones_v, sums_sh, cnt_sh,
              sem_a, sem_b):
    core = lax.axis_index("c")
    sid = lax.axis_index("s")
    vrpt = vpad // NS
    cat_c = cat_hbm.at[core]
    g_c = g_hbm.at[core]

    _fill(zbuf96, 0.0)
    _fill(cnt_buf, 0.0)
    _fill(ones_v, 1.0)
    _zero_shared(sums_sh, zbuf96, sid * vrpt, vrpt // ECHUNK)
    _zero_shared(cnt_sh, cnt_buf, sid * vrpt, vrpt // ECHUNK)
    pltpu.sync_copy(vox_hbm.at[sid], vox_v)
    plsc.subcore_barrier()

    def rows(j):
        return pl.ds((sid * pch + j) * ECHUNK, ECHUNK)

    def issue(j, buf, sem):
        pltpu.async_copy(cat_c.at[rows(j)], buf, sem)

    def wait(j, buf, sem):
        pltpu.make_async_copy(cat_c.at[rows(j)], buf, sem).wait()

    def scat(j, buf):
        pltpu.sync_copy(buf, sums_sh.at[vox_v.at[j]], add=True)
        pltpu.sync_copy(ones_v, cnt_sh.at[vox_v.at[j]], add=True)

    issue(0, buf_a, sem_a)

    @pl.loop(0, pch, step=2)
    def _(j):
        issue(j + 1, buf_b, sem_b)
        wait(j, buf_a, sem_a)
        scat(j, buf_a)

        @pl.when(j + 2 < pch)
        def _():
            issue(j + 2, buf_a, sem_a)

        wait(j + 1, buf_b, sem_b)
        scat(j + 1, buf_b)

    plsc.subcore_barrier()

    @pl.loop(0, pch)
    def _(j):
        pltpu.sync_copy(sums_sh.at[vox_v.at[j]], buf_a)
        pltpu.sync_copy(buf_a, g_c.at[rows(j)])
        pltpu.sync_copy(cnt_sh.at[vox_v.at[j]], cnt_buf)
        pltpu.sync_copy(cnt_buf, gcnt_hbm.at[rows(j)])


def _run_scc(cat_all, voxp, vpad):
    npt = cat_all.shape[1]
    pch = voxp.shape[1]
    mesh = plsc.VectorSubcoreMesh(core_axis_name="c", subcore_axis_name="s")
    kern = pl.kernel(
        functools.partial(_scc_body, pch, vpad),
        out_type=[jax.ShapeDtypeStruct((2, npt, 96), jnp.float32),
                  jax.ShapeDtypeStruct((npt, 16), jnp.float32)],
        mesh=mesh,
        scratch_types=[
            pltpu.VMEM((pch, ECHUNK), jnp.int32),
            pltpu.VMEM((ECHUNK, 96), jnp.float32),
            pltpu.VMEM((ECHUNK, 96), jnp.float32),
            pltpu.VMEM((ECHUNK, 16), jnp.float32),
            pltpu.VMEM((ECHUNK, 96), jnp.float32),
            pltpu.VMEM((ECHUNK, 16), jnp.float32),
            pltpu.VMEM_SHARED((vpad, 96), jnp.float32),
            pltpu.VMEM_SHARED((vpad, 16), jnp.float32),
            pltpu.SemaphoreType.DMA,
            pltpu.SemaphoreType.DMA,
        ],
        compiler_params=_SC_PARAMS,
    )
    return kern(cat_all, voxp)


def _tcd_body(eps, gall_ref, gcnt_ref, pwc_ref, pws_ref, cat1_ref,
              agg_ref, deg_ref, wl_ref, ng_ref, nb_ref, lg_ref,
              lb_ref, out_ref):
    inv = 1.0 / jnp.maximum(gcnt_ref[...][:, :1], 1.0)
    g0 = gall_ref[0]
    g1 = gall_ref[1]
    gcos = g0[:, :64] * inv
    gsin = jnp.concatenate([g0[:, 64:], g1[:, :32]], axis=1) * inv
    glin = g1[:, 32:] * inv
    fwl = cat1_ref[0][:, 32:]
    new_f = gcos * pwc_ref[...] + gsin * pws_ref[...] + (glin - fwl)

    a = agg_ref[...]
    agg = jnp.concatenate([a[0], a[1], a[2], a[3]], axis=1)
    loc = jnp.dot(agg / jnp.maximum(deg_ref[...][:, :1], 1.0), wl_ref[...],
                  preferred_element_type=jnp.float32)

    def ln(x, g, b):
        m = jnp.mean(x, axis=-1, keepdims=True)
        v = jnp.mean((x - m) ** 2, axis=-1, keepdims=True)
        return (x - m) / jnp.sqrt(v + eps) * g + b

    out_ref[...] = jax.nn.relu(ln(new_f, ng_ref[...], nb_ref[...])
                               + ln(loc, lg_ref[...], lb_ref[...]))


def _run_tcd(n, g_all, gcnt, pwc, pws, cat_all, agg_all, deg,
             W_local, norm_g, norm_b, nl_g, nl_b):
    blk = 2000
    grid = (n // blk,)
    row_spec = lambda w: pl.BlockSpec((blk, w), lambda i: (i, 0))
    full2 = lambda a: pl.BlockSpec(a.shape, lambda i: (0,) * a.ndim)
    return pl.pallas_call(
        functools.partial(_tcd_body, 1e-6),
        grid=grid,
        in_specs=[pl.BlockSpec((2, blk, 96), lambda i: (0, i, 0)),
                  row_spec(16), row_spec(64), row_spec(64),
                  pl.BlockSpec((1, blk, 96), lambda i: (1, i, 0)),
                  pl.BlockSpec((4, blk, 16), lambda i: (0, i, 0)),
                  row_spec(16), full2(W_local), full2(norm_g), full2(norm_b),
                  full2(nl_g), full2(nl_b)],
        out_specs=row_spec(64),
        out_shape=jax.ShapeDtypeStruct((n, 64), jnp.float32),
    )(g_all, gcnt, pwc, pws, cat_all, agg_all, deg,
      W_local, norm_g, norm_b, nl_g, nl_b)


def kernel(F, C, edge_index, voxel_idx, W_pre, ln_pre_g, ln_pre_b, W_pos,
           alpha, W_local, norm_g, norm_b, nl_g, nl_b):
    n, inc = F.shape
    e = edge_index.shape[1]
    nvox = 6250

    e_pad = _ceil_to(e, NS * IBLK * ECHUNK)
    ept = e_pad // NS
    nch = ept // ECHUNK
    npad = _ceil_to(n + 1, NS * ECHUNK)
    pch = _ceil_to(_ceil_to(n, NS * ECHUNK) // (NS * ECHUNK), 2)
    npt = NS * pch * ECHUNK
    vpad = _ceil_to(nvox + 1, NS * ECHUNK)

    Fp = jnp.pad(F, ((0, npt - n), (0, 0)))
    Cp = jnp.pad(C, ((0, npt - n), (0, 5)))
    W_posp = jnp.pad(W_pos, ((0, 5), (0, 0)))
    fq = jnp.transpose(F.reshape(n, 4, 16), (1, 0, 2))
    srcp = jnp.concatenate(
        [edge_index[0], jnp.zeros((e_pad - e,), jnp.int32)]).reshape(
            NS, nch, ECHUNK)
    dstp = jnp.concatenate(
        [edge_index[1], jnp.full((e_pad - e,), n, jnp.int32)]).reshape(
            NS, nch, ECHUNK)
    voxp = jnp.concatenate(
        [voxel_idx, jnp.full((npt - n,), nvox, jnp.int32)]).reshape(
            NS, pch, ECHUNK)

    cat_all, pwc, pws = _run_tca(Fp, Cp, W_pre, ln_pre_g, ln_pre_b,
                                 W_posp, alpha)
    agg_all, deg = _run_scb(fq, srcp, dstp, npad)
    voxp = voxp + deg[:1, :1].astype(jnp.int32) * 0
    g_all, gcnt = _run_scc(cat_all, voxp, vpad)
    out = _run_tcd(n, g_all, gcnt, pwc, pws, cat_all, agg_all, deg,
                   W_local, norm_g, norm_b, nl_g, nl_b)
    return out


# speedup vs baseline: 7.2434x; 1.4742x over previous
import functools

import jax
import jax.numpy as jnp
from jax import lax
from jax.experimental import pallas as pl
from jax.experimental.pallas import tpu as pltpu
from jax.experimental.pallas import tpu_sc as plsc

NC, NS = 2, 16
ECHUNK = 128

_SC_PARAMS = pltpu.CompilerParams(use_tc_tiling_on_sc=False)


def _ceil_to(x, m):
    return (x + m - 1) // m * m


def _tca_body(eps, f_ref, c_ref, wpre_ref, g_ref, b_ref, wpos_ref, alpha_ref,
              cat_ref, pwc_ref, pws_ref):
    x = jnp.dot(f_ref[...], wpre_ref[...], preferred_element_type=jnp.float32)
    m = jnp.mean(x, axis=-1, keepdims=True)
    v = jnp.mean((x - m) ** 2, axis=-1, keepdims=True)
    fi = (x - m) / jnp.sqrt(v + eps) * g_ref[...] + b_ref[...]
    pw = jnp.dot(c_ref[...], wpos_ref[...],
                 preferred_element_type=jnp.float32) * alpha_ref[...]
    s = jnp.sin(pw)
    co = jnp.cos(pw)
    fc = fi * co
    fs = fi * s
    fl = fi * pw
    cat_ref[0] = jnp.concatenate([fc, fs[:, :32]], axis=1)
    cat_ref[1] = jnp.concatenate([fs[:, 32:], fl], axis=1)
    pwc_ref[...] = co
    pws_ref[...] = s


def _run_tca(Fp, Cp, W_pre, ln_g, ln_b, W_posp, alpha):
    npt, inc = Fp.shape
    blk = 1024
    grid = (npt // blk,)
    row_spec = lambda w: pl.BlockSpec((blk, w), lambda i: (i, 0))
    full2 = lambda a: pl.BlockSpec(a.shape, lambda i: (0,) * a.ndim)
    return pl.pallas_call(
        functools.partial(_tca_body, 1e-6),
        grid=grid,
        in_specs=[row_spec(inc), row_spec(Cp.shape[1]), full2(W_pre),
                  full2(ln_g), full2(ln_b), full2(W_posp), full2(alpha)],
        out_specs=[pl.BlockSpec((2, blk, 96), lambda i: (0, i, 0)),
                   row_spec(inc), row_spec(inc)],
        out_shape=[jax.ShapeDtypeStruct((2, npt, 96), jnp.float32),
                   jax.ShapeDtypeStruct((npt, inc), jnp.float32),
                   jax.ShapeDtypeStruct((npt, inc), jnp.float32)],
    )(Fp, Cp, W_pre, ln_g, ln_b, W_posp, alpha)


def _fill(buf, value):
    n, w = buf.shape

    @pl.loop(0, n)
    def _(i):
        row = buf.at[i]
        for k in range(w // 16):
            row[pl.ds(k * 16, 16)] = jnp.full((16,), value, jnp.float32)


def _zero_shared(sh, zbuf, base, nchunks):
    @pl.loop(0, nchunks)
    def _(i):
        pltpu.sync_copy(zbuf, sh.at[pl.ds(base + i * ECHUNK, ECHUNK)])


GCH = 14
NGRP = 4
IBLK = GCH * NGRP


def _scb_body(nch, npad,
              fq_hbm, src_hbm, dst_hbm,
              agg_hbm, deg_hbm,
              src_v, dst_v, bank_a, bank_b, zbuf, agg_sh,
              sem_ga, sem_gb, sem_sa, sem_sb):
    core = lax.axis_index("c")
    sid = lax.axis_index("s")
    rpt = npad // NS
    zslc = pl.ds(sid * rpt, rpt)
    nblk = nch // IBLK
    src_t = src_hbm.at[sid]
    dst_t = dst_hbm.at[sid]
    grp = lambda g: pl.ds(g * GCH * ECHUNK, GCH * ECHUNK)
    blk = lambda b: pl.ds(b * IBLK * ECHUNK, IBLK * ECHUNK)

    _fill(zbuf, 0.0)

    for p in range(2):
        q = 2 * p + core
        f_hbm = fq_hbm.at[q]
        _zero_shared(agg_sh, zbuf, sid * rpt, rpt // ECHUNK)
        plsc.subcore_barrier()

        def issue_g(g, buf, sem):
            pltpu.async_copy(f_hbm.at[src_v.at[grp(g)]], buf, sem)

        def wait_g(g, buf, sem):
            pltpu.make_async_copy(f_hbm.at[src_v.at[grp(g)]], buf, sem).wait()

        def issue_s(g, buf, sem):
            pltpu.async_copy(buf, agg_sh.at[dst_v.at[grp(g)]], sem, add=True)

        def wait_s(g, buf, sem):
            pltpu.make_async_copy(buf, agg_sh.at[dst_v.at[grp(g)]],
                                  sem).wait()

        @pl.loop(0, nblk)
        def _(b):
            pltpu.sync_copy(src_t.at[blk(b)], src_v)
            pltpu.sync_copy(dst_t.at[blk(b)], dst_v)
            issue_g(0, bank_a, sem_ga)
            issue_g(1, bank_b, sem_gb)
            for g in range(0, NGRP, 2):
                wait_g(g, bank_a, sem_ga)
                issue_s(g, bank_a, sem_sa)
                wait_g(g + 1, bank_b, sem_gb)
                issue_s(g + 1, bank_b, sem_sb)
                if g + 2 < NGRP:
                    wait_s(g, bank_a, sem_sa)
                    issue_g(g + 2, bank_a, sem_ga)
                    wait_s(g + 1, bank_b, sem_sb)
                    issue_g(g + 3, bank_b, sem_gb)
            wait_s(NGRP - 2, bank_a, sem_sa)
            wait_s(NGRP - 1, bank_b, sem_sb)

        plsc.subcore_barrier()
        pltpu.sync_copy(agg_sh.at[zslc], agg_hbm.at[q].at[zslc])

    _zero_shared(agg_sh, zbuf, sid * rpt, rpt // ECHUNK)
    _fill(bank_a, 1.0)
    plsc.subcore_barrier()

    @pl.loop(0, nblk)
    def _(b):
        pltpu.sync_copy(dst_t.at[blk(b)], dst_v)
        for g in range(NGRP):
            pltpu.async_copy(bank_a, agg_sh.at[dst_v.at[grp(g)]], sem_sa,
                             add=True)
        for g in range(NGRP):
            pltpu.make_async_copy(bank_a, agg_sh.at[dst_v.at[grp(g)]],
                                  sem_sa).wait()

    plsc.subcore_barrier()
    pltpu.sync_copy(agg_sh.at[zslc], deg_hbm.at[zslc])


def _run_scb(fq, srcp, dstp, npad):
    nch = srcp.shape[1] // ECHUNK
    mesh = plsc.VectorSubcoreMesh(core_axis_name="c", subcore_axis_name="s")
    kern = pl.kernel(
        functools.partial(_scb_body, nch, npad),
        out_type=[jax.ShapeDtypeStruct((4, npad, 16), jnp.float32),
                  jax.ShapeDtypeStruct((npad, 16), jnp.float32)],
        mesh=mesh,
        scratch_types=[
            pltpu.VMEM((IBLK * ECHUNK,), jnp.int32),
            pltpu.VMEM((IBLK * ECHUNK,), jnp.int32),
            pltpu.VMEM((GCH * ECHUNK, 16), jnp.float32),
            pltpu.VMEM((GCH * ECHUNK, 16), jnp.float32),
            pltpu.VMEM((ECHUNK, 16), jnp.float32),
            pltpu.VMEM_SHARED((npad, 16), jnp.float32),
            pltpu.SemaphoreType.DMA,
            pltpu.SemaphoreType.DMA,
            pltpu.SemaphoreType.DMA,
            pltpu.SemaphoreType.DMA,
        ],
        compiler_params=_SC_PARAMS,
    )
    return kern(fq, srcp, dstp)


def _scc_body(pch, vpad,
              cat_hbm, vox_hbm,
              g_hbm, gcnt_hbm,
              vox_v, buf_a, buf_b, cnt_buf, zbuf96, ones_v, sums_sh, cnt_sh,
              sem_a, sem_b):
    core = lax.axis_index("c")
    sid = lax.axis_index("s")
    vrpt = vpad // NS
    cat_c = cat_hbm.at[core]
    g_c = g_hbm.at[core]

    _fill(zbuf96, 0.0)
    _fill(cnt_buf, 0.0)
    _fill(ones_v, 1.0)
    _zero_shared(sums_sh, zbuf96, sid * vrpt, vrpt // ECHUNK)
    _zero_shared(cnt_sh, cnt_buf, sid * vrpt, vrpt // ECHUNK)
    pltpu.sync_copy(vox_hbm.at[sid], vox_v)
    plsc.subcore_barrier()

    def rows(j):
        return pl.ds((sid * pch + j) * ECHUNK, ECHUNK)

    def issue(j, buf, sem):
        pltpu.async_copy(cat_c.at[rows(j)], buf, sem)

    def wait(j, buf, sem):
        pltpu.make_async_copy(cat_c.at[rows(j)], buf, sem).wait()

    def scat(j, buf):
        pltpu.sync_copy(buf, sums_sh.at[vox_v.at[j]], add=True)
        pltpu.sync_copy(ones_v, cnt_sh.at[vox_v.at[j]], add=True)

    issue(0, buf_a, sem_a)

    @pl.loop(0, pch, step=2)
    def _(j):
        issue(j + 1, buf_b, sem_b)
        wait(j, buf_a, sem_a)
        scat(j, buf_a)

        @pl.when(j + 2 < pch)
        def _():
            issue(j + 2, buf_a, sem_a)

        wait(j + 1, buf_b, sem_b)
        scat(j + 1, buf_b)

    plsc.subcore_barrier()

    @pl.loop(0, pch)
    def _(j):
        pltpu.sync_copy(sums_sh.at[vox_v.at[j]], buf_a)
        pltpu.sync_copy(buf_a, g_c.at[rows(j)])
        pltpu.sync_copy(cnt_sh.at[vox_v.at[j]], cnt_buf)
        pltpu.sync_copy(cnt_buf, gcnt_hbm.at[rows(j)])


def _run_scc(cat_all, voxp, vpad):
    npt = cat_all.shape[1]
    pch = voxp.shape[1]
    mesh = plsc.VectorSubcoreMesh(core_axis_name="c", subcore_axis_name="s")
    kern = pl.kernel(
        functools.partial(_scc_body, pch, vpad),
        out_type=[jax.ShapeDtypeStruct((2, npt, 96), jnp.float32),
                  jax.ShapeDtypeStruct((npt, 16), jnp.float32)],
        mesh=mesh,
        scratch_types=[
            pltpu.VMEM((pch, ECHUNK), jnp.int32),
            pltpu.VMEM((ECHUNK, 96), jnp.float32),
            pltpu.VMEM((ECHUNK, 96), jnp.float32),
            pltpu.VMEM((ECHUNK, 16), jnp.float32),
            pltpu.VMEM((ECHUNK, 96), jnp.float32),
            pltpu.VMEM((ECHUNK, 16), jnp.float32),
            pltpu.VMEM_SHARED((vpad, 96), jnp.float32),
            pltpu.VMEM_SHARED((vpad, 16), jnp.float32),
            pltpu.SemaphoreType.DMA,
            pltpu.SemaphoreType.DMA,
        ],
        compiler_params=_SC_PARAMS,
    )
    return kern(cat_all, voxp)


def _tcd_body(eps, gall_ref, gcnt_ref, pwc_ref, pws_ref, cat1_ref,
              agg_ref, deg_ref, wl_ref, ng_ref, nb_ref, lg_ref,
              lb_ref, out_ref):
    inv = 1.0 / jnp.maximum(gcnt_ref[...][:, :1], 1.0)
    g0 = gall_ref[0]
    g1 = gall_ref[1]
    gcos = g0[:, :64] * inv
    gsin = jnp.concatenate([g0[:, 64:], g1[:, :32]], axis=1) * inv
    glin = g1[:, 32:] * inv
    fwl = cat1_ref[0][:, 32:]
    new_f = gcos * pwc_ref[...] + gsin * pws_ref[...] + (glin - fwl)

    a = agg_ref[...]
    agg = jnp.concatenate([a[0], a[1], a[2], a[3]], axis=1)
    loc = jnp.dot(agg / jnp.maximum(deg_ref[...][:, :1], 1.0), wl_ref[...],
                  preferred_element_type=jnp.float32)

    def ln(x, g, b):
        m = jnp.mean(x, axis=-1, keepdims=True)
        v = jnp.mean((x - m) ** 2, axis=-1, keepdims=True)
        return (x - m) / jnp.sqrt(v + eps) * g + b

    out_ref[...] = jax.nn.relu(ln(new_f, ng_ref[...], nb_ref[...])
                               + ln(loc, lg_ref[...], lb_ref[...]))


def _run_tcd(n, g_all, gcnt, pwc, pws, cat_all, agg_all, deg,
             W_local, norm_g, norm_b, nl_g, nl_b):
    blk = 2000
    grid = (n // blk,)
    row_spec = lambda w: pl.BlockSpec((blk, w), lambda i: (i, 0))
    full2 = lambda a: pl.BlockSpec(a.shape, lambda i: (0,) * a.ndim)
    return pl.pallas_call(
        functools.partial(_tcd_body, 1e-6),
        grid=grid,
        in_specs=[pl.BlockSpec((2, blk, 96), lambda i: (0, i, 0)),
                  row_spec(16), row_spec(64), row_spec(64),
                  pl.BlockSpec((1, blk, 96), lambda i: (1, i, 0)),
                  pl.BlockSpec((4, blk, 16), lambda i: (0, i, 0)),
                  row_spec(16), full2(W_local), full2(norm_g), full2(norm_b),
                  full2(nl_g), full2(nl_b)],
        out_specs=row_spec(64),
        out_shape=jax.ShapeDtypeStruct((n, 64), jnp.float32),
    )(g_all, gcnt, pwc, pws, cat_all, agg_all, deg,
      W_local, norm_g, norm_b, nl_g, nl_b)


def kernel(F, C, edge_index, voxel_idx, W_pre, ln_pre_g, ln_pre_b, W_pos,
           alpha, W_local, norm_g, norm_b, nl_g, nl_b):
    n, inc = F.shape
    e = edge_index.shape[1]
    nvox = 6250

    e_pad = _ceil_to(e, NS * IBLK * ECHUNK)
    ept = e_pad // NS
    nch = ept // ECHUNK
    npad = _ceil_to(n + 1, NS * ECHUNK)
    pch = _ceil_to(_ceil_to(n, NS * ECHUNK) // (NS * ECHUNK), 2)
    npt = NS * pch * ECHUNK
    vpad = _ceil_to(nvox + 1, NS * ECHUNK)

    Fp = jnp.pad(F, ((0, npt - n), (0, 0)))
    Cp = jnp.pad(C, ((0, npt - n), (0, 5)))
    W_posp = jnp.pad(W_pos, ((0, 5), (0, 0)))
    fq = jnp.transpose(F.reshape(n, 4, 16), (1, 0, 2))
    srcp = jnp.concatenate(
        [edge_index[0], jnp.zeros((e_pad - e,), jnp.int32)]).reshape(
            NS, nch * ECHUNK)
    dstp = jnp.concatenate(
        [edge_index[1], jnp.full((e_pad - e,), n, jnp.int32)]).reshape(
            NS, nch * ECHUNK)
    voxp = jnp.concatenate(
        [voxel_idx, jnp.full((npt - n,), nvox, jnp.int32)]).reshape(
            NS, pch, ECHUNK)

    cat_all, pwc, pws = _run_tca(Fp, Cp, W_pre, ln_pre_g, ln_pre_b,
                                 W_posp, alpha)
    agg_all, deg = _run_scb(fq, srcp, dstp, npad)
    voxp, _ = lax.optimization_barrier((voxp, deg))
    g_all, gcnt = _run_scc(cat_all, voxp, vpad)
    out = _run_tcd(n, g_all, gcnt, pwc, pws, cat_all, agg_all, deg,
                   W_local, norm_g, norm_b, nl_g, nl_b)
    return out
